# Initial kernel scaffold; baseline (speedup 1.0000x reference)
import jax
import jax.numpy as jnp
from jax.experimental import pallas as pl


def kernel(cart, shifts, species, radial_params, emb_params, mp_params, out_params, atomindex):
    raise NotImplementedError("write your pallas kernel here")


# speedup vs baseline: 18.3732x; 18.3732x over previous
import functools

import jax
import jax.numpy as jnp
import numpy as np
from jax import lax
from jax.experimental import pallas as pl
from jax.experimental.pallas import tpu as pltpu
from jax.experimental.pallas import tpu_sc as plsc

N_ATOMS = 10000
N_EDGES = 320000
NWAVE = 8
NSPH = 9
RL = 3
NORBIT = 24
CUTOFF = 5.0
MP_LOOP = 2

A_BLK = 128
N_PAD = 10240
NB = N_PAD // A_BLK
K_WIN = 512
E_BLK = 2000
D_TAB = 128
D_CART = 128
M_BLK = 1024

_R = np.zeros((NSPH, 72), np.float32)
_T = np.zeros((NWAVE, 72), np.float32)
_P = np.zeros((72, NORBIT), np.float32)
_L_OF_S = [0, 1, 1, 1, 2, 2, 2, 2, 2]
for s in range(NSPH):
    for w in range(NWAVE):
        _R[s, s * 8 + w] = 1.0
        _T[w, s * 8 + w] = 1.0
        _P[s * 8 + w, _L_OF_S[s] * 8 + w] = 1.0


def _sc_gather(table, idx, d):
    info = plsc.get_sparse_core_info()
    nc, ns = info.num_cores, info.num_subcores
    nw = nc * ns
    b = idx.shape[0]
    b_per_w = b // nw
    chunk = 80
    n_chunks = b_per_w // chunk
    mesh = plsc.VectorSubcoreMesh(core_axis_name="c", subcore_axis_name="s")

    @functools.partial(
        pl.kernel, mesh=mesh,
        out_type=jax.ShapeDtypeStruct((b, d), jnp.float32),
        scratch_types=[
            pltpu.VMEM((chunk,), jnp.int32),
            pltpu.VMEM((chunk, d), jnp.float32),
            pltpu.SemaphoreType.DMA,
        ],
    )
    def k(table_hbm, idx_hbm, out_hbm, idx_v, rows_v, sem):
        wid = lax.axis_index("s") * nc + lax.axis_index("c")
        base = wid * b_per_w

        def body(t, carry):
            off = base + t * chunk
            pltpu.sync_copy(idx_hbm.at[pl.ds(off, chunk)], idx_v)
            pltpu.async_copy(table_hbm.at[idx_v], rows_v, sem).wait()
            pltpu.sync_copy(rows_v, out_hbm.at[pl.ds(off, chunk)])
            return carry

        lax.fori_loop(0, n_chunks, body, 0)

    return k(table, idx)


def _geom_kernel(cn, cc, sh, alpha, rs):
    def body(cn_ref, cc_ref, sh_ref, al_ref, rs_ref, rad_ref, sph_ref):
        c = cn_ref[:, 0:3] - cc_ref[:, 0:3] + sh_ref[...]
        x = c[:, 0:1] / CUTOFF
        y = c[:, 1:2] / CUTOFF
        z = c[:, 2:3] / CUTOFF
        r2 = x * x + y * y + z * z
        d = jnp.sqrt(r2) * CUTOFF
        fc = 0.5 * (jnp.cos(np.pi * jnp.clip(d, 0.0, CUTOFF) / CUTOFF) + 1.0)
        g = jnp.exp(-jnp.abs(al_ref[...]) * (d - rs_ref[...]) ** 2)
        rad_ref[...] = g * fc
        sph_ref[...] = jnp.concatenate(
            [jnp.ones_like(x), y, z, x, x * y, y * z, 3.0 * z * z - r2,
             x * z, x * x - y * y], axis=1)

    e = cn.shape[0]
    grid = e // E_BLK
    return pl.pallas_call(
        body,
        grid=(grid,),
        in_specs=[
            pl.BlockSpec((E_BLK, D_CART), lambda i: (i, 0)),
            pl.BlockSpec((E_BLK, D_CART), lambda i: (i, 0)),
            pl.BlockSpec((E_BLK, 3), lambda i: (i, 0)),
            pl.BlockSpec((1, NWAVE), lambda i: (0, 0)),
            pl.BlockSpec((1, NWAVE), lambda i: (0, 0)),
        ],
        out_specs=[
            pl.BlockSpec((E_BLK, NWAVE), lambda i: (i, 0)),
            pl.BlockSpec((E_BLK, NSPH), lambda i: (i, 0)),
        ],
        out_shape=[
            jax.ShapeDtypeStruct((e, NWAVE), jnp.float32),
            jax.ShapeDtypeStruct((e, NSPH), jnp.float32),
        ],
    )(cn, cc, sh, alpha, rs)


def _msg_kernel(rows, rad, sph):
    sel_m = np.zeros((D_TAB, 72), np.float32)
    sel_m[0:72, 0:72] = np.eye(72, dtype=np.float32)
    sel_ct = np.zeros((D_TAB, 72), np.float32)
    for s in range(NSPH):
        for w in range(NWAVE):
            sel_ct[72 + w, s * 8 + w] = 1.0

    def body(rows_ref, rad_ref, sph_ref, r_ref, t_ref, sm_ref, sct_ref,
             out_ref):
        rows_v = rows_ref[...]
        mp72 = jnp.dot(rows_v, sm_ref[...], preferred_element_type=jnp.float32)
        c72 = jnp.dot(rows_v, sct_ref[...], preferred_element_type=jnp.float32)
        s72 = jnp.dot(sph_ref[...], r_ref[...],
                      preferred_element_type=jnp.float32)
        r72 = jnp.dot(rad_ref[...], t_ref[...],
                      preferred_element_type=jnp.float32)
        out_ref[...] = (s72 + mp72) * (r72 * c72)

    e = rows.shape[0]
    grid = e // E_BLK
    return pl.pallas_call(
        body,
        grid=(grid,),
        in_specs=[
            pl.BlockSpec((E_BLK, D_TAB), lambda i: (i, 0)),
            pl.BlockSpec((E_BLK, NWAVE), lambda i: (i, 0)),
            pl.BlockSpec((E_BLK, NSPH), lambda i: (i, 0)),
            pl.BlockSpec((NSPH, 72), lambda i: (0, 0)),
            pl.BlockSpec((NWAVE, 72), lambda i: (0, 0)),
            pl.BlockSpec((D_TAB, 72), lambda i: (0, 0)),
            pl.BlockSpec((D_TAB, 72), lambda i: (0, 0)),
        ],
        out_specs=pl.BlockSpec((E_BLK, 72), lambda i: (i, 0)),
        out_shape=jax.ShapeDtypeStruct((e, 72), jnp.float32),
    )(rows, rad, sph, jnp.asarray(_R), jnp.asarray(_T),
      jnp.asarray(sel_m), jnp.asarray(sel_ct))


def _scatter_kernel(starts, sc_row, worbit):
    def body(starts_ref, sc_hbm, wo_hbm, out_ref, sc_v, wo_v, acc_ref, s1, s2):
        b = pl.program_id(0)
        e0 = starts_ref[b]
        e1 = starts_ref[b + 1]
        t0 = e0 // K_WIN
        t1 = lax.div(e1 + K_WIN - 1, K_WIN)
        rows = lax.broadcasted_iota(jnp.int32, (A_BLK, K_WIN), 0) + b * A_BLK
        acc_ref[...] = jnp.zeros((A_BLK, 72), jnp.float32)

        def win(t, carry):
            cp1 = pltpu.make_async_copy(
                sc_hbm.at[pl.ds(0, 1), pl.ds(t * K_WIN, K_WIN)], sc_v, s1)
            cp2 = pltpu.make_async_copy(
                wo_hbm.at[pl.ds(t * K_WIN, K_WIN), :], wo_v, s2)
            cp1.start()
            cp2.start()
            cp1.wait()
            cp2.wait()
            onehot = jnp.where(rows == sc_v[...], 1.0, 0.0)
            acc_ref[...] += jnp.dot(onehot, wo_v[...],
                                    preferred_element_type=jnp.float32)
            return carry

        lax.fori_loop(t0, t1, win, 0)
        out_ref[...] = acc_ref[...]

    grid_spec = pltpu.PrefetchScalarGridSpec(
        num_scalar_prefetch=1,
        grid=(NB,),
        in_specs=[
            pl.BlockSpec(memory_space=pl.ANY),
            pl.BlockSpec(memory_space=pl.ANY),
        ],
        out_specs=pl.BlockSpec((A_BLK, 72), lambda b, s: (b, 0)),
        scratch_shapes=[
            pltpu.VMEM((1, K_WIN), jnp.int32),
            pltpu.VMEM((K_WIN, 72), jnp.float32),
            pltpu.VMEM((A_BLK, 72), jnp.float32),
            pltpu.SemaphoreType.DMA,
            pltpu.SemaphoreType.DMA,
        ],
    )
    return pl.pallas_call(
        body,
        grid_spec=grid_spec,
        out_shape=jax.ShapeDtypeStruct((N_PAD, 72), jnp.float32),
    )(starts, sc_row, worbit)


def _mlp3(x_ref, w1_ref, b1_ref, w2_ref, b2_ref, w3_ref, b3_ref):
    h = jnp.dot(x_ref[...], w1_ref[...], preferred_element_type=jnp.float32)
    h = jax.nn.silu(h + b1_ref[...])
    h = jnp.dot(h, w2_ref[...], preferred_element_type=jnp.float32)
    h = jax.nn.silu(h + b2_ref[...])
    h = jnp.dot(h, w3_ref[...], preferred_element_type=jnp.float32)
    return h + b3_ref[...]


def _mlp_specs(din, dout):
    return [
        pl.BlockSpec((din, 64), lambda i: (0, 0)),
        pl.BlockSpec((1, 64), lambda i: (0, 0)),
        pl.BlockSpec((64, 64), lambda i: (0, 0)),
        pl.BlockSpec((1, 64), lambda i: (0, 0)),
        pl.BlockSpec((64, dout), lambda i: (0, 0)),
        pl.BlockSpec((1, dout), lambda i: (0, 0)),
    ]


def _emb_kernel(species_pad, params):
    def body(x_ref, w1, b1, w2, b2, w3, b3, out_ref):
        out_ref[...] = _mlp3(x_ref, w1, b1, w2, b2, w3, b3)

    (w1, b1), (w2, b2), (w3, b3) = params
    return pl.pallas_call(
        body,
        grid=(N_PAD // M_BLK,),
        in_specs=[pl.BlockSpec((M_BLK, 1), lambda i: (i, 0))]
        + _mlp_specs(1, NWAVE),
        out_specs=pl.BlockSpec((M_BLK, NWAVE), lambda i: (i, 0)),
        out_shape=jax.ShapeDtypeStruct((N_PAD, NWAVE), jnp.float32),
    )(species_pad, w1, b1[None, :], w2, b2[None, :], w3, b3[None, :])


def _dens_mlp_kernel(orbit, dens, params):
    def body(o_ref, d_ref, p_ref, w1, b1, w2, b2, w3, b3, dn_ref, c_ref):
        o = o_ref[...]
        dn = d_ref[...] + jnp.dot(o * o, p_ref[...],
                                  preferred_element_type=jnp.float32)
        dn_ref[...] = dn
        c_ref[...] = _mlp3(dn_ref, w1, b1, w2, b2, w3, b3)

    (w1, b1), (w2, b2), (w3, b3) = params
    return pl.pallas_call(
        body,
        grid=(N_PAD // M_BLK,),
        in_specs=[
            pl.BlockSpec((M_BLK, 72), lambda i: (i, 0)),
            pl.BlockSpec((M_BLK, NORBIT), lambda i: (i, 0)),
            pl.BlockSpec((72, NORBIT), lambda i: (0, 0)),
        ] + _mlp_specs(NORBIT, NWAVE),
        out_specs=[
            pl.BlockSpec((M_BLK, NORBIT), lambda i: (i, 0)),
            pl.BlockSpec((M_BLK, NWAVE), lambda i: (i, 0)),
        ],
        out_shape=[
            jax.ShapeDtypeStruct((N_PAD, NORBIT), jnp.float32),
            jax.ShapeDtypeStruct((N_PAD, NWAVE), jnp.float32),
        ],
    )(orbit, dens, jnp.asarray(_P), w1, b1[None, :], w2, b2[None, :],
      w3, b3[None, :])


def _final_kernel(orbit, dens, params):
    def body(o_ref, d_ref, p_ref, w1, b1, w2, b2, w3, b3, out_ref, dn_ref):
        i = pl.program_id(0)
        o = o_ref[...]
        dn_ref[...] = d_ref[...] + jnp.dot(
            o * o, p_ref[...], preferred_element_type=jnp.float32)
        y = _mlp3(dn_ref, w1, b1, w2, b2, w3, b3)
        row = lax.broadcasted_iota(jnp.int32, (M_BLK, 1), 0) + i * M_BLK
        y = jnp.where(row < N_ATOMS, y, 0.0)
        s = jnp.sum(y, axis=0, keepdims=True)

        @pl.when(i == 0)
        def _():
            out_ref[...] = jnp.zeros((1, 1), jnp.float32)

        out_ref[...] += s

    (w1, b1), (w2, b2), (w3, b3) = params
    out, _ = pl.pallas_call(
        body,
        grid=(N_PAD // M_BLK,),
        in_specs=[
            pl.BlockSpec((M_BLK, 72), lambda i: (i, 0)),
            pl.BlockSpec((M_BLK, NORBIT), lambda i: (i, 0)),
            pl.BlockSpec((72, NORBIT), lambda i: (0, 0)),
        ] + _mlp_specs(NORBIT, 1),
        out_specs=[
            pl.BlockSpec((1, 1), lambda i: (0, 0)),
            pl.BlockSpec((M_BLK, NORBIT), lambda i: (i, 0)),
        ],
        out_shape=[
            jax.ShapeDtypeStruct((1, 1), jnp.float32),
            jax.ShapeDtypeStruct((N_PAD, NORBIT), jnp.float32),
        ],
    )(orbit, dens, jnp.asarray(_P), w1, b1[None, :], w2, b2[None, :],
      w3, b3[None, :])
    return out[0, 0]


def kernel(cart, shifts, species, radial_params, emb_params, mp_params,
           out_params, atomindex):
    idx_c = atomindex[0]
    idx_n = atomindex[1]

    perm = jnp.argsort(idx_c)
    sc = idx_c[perm]
    sn = idx_n[perm]
    sh = shifts[:, perm].T
    starts = jnp.searchsorted(
        sc, jnp.arange(NB + 1, dtype=jnp.int32) * A_BLK).astype(jnp.int32)
    sc_row = sc[None, :]

    cart_t = jnp.zeros((N_ATOMS, D_CART), jnp.float32)
    cart_t = cart_t.at[:, 0:3].set(cart.T)

    cn = _sc_gather(cart_t, sn, D_CART)
    cc = _sc_gather(cart_t, sc, D_CART)
    alpha, rs = radial_params
    rad, sph = _geom_kernel(cn, cc, sh, alpha[None, :], rs[None, :])

    species_pad = jnp.zeros((N_PAD, 1), jnp.float32).at[:N_ATOMS].set(species)
    coeff = _emb_kernel(species_pad, emb_params)
    dens = jnp.zeros((N_PAD, NORBIT), jnp.float32)
    orbit = jnp.zeros((N_PAD, 72), jnp.float32)

    for r in range(MP_LOOP + 1):
        table = jnp.concatenate(
            [orbit[:N_ATOMS], coeff[:N_ATOMS],
             jnp.zeros((N_ATOMS, D_TAB - 80), jnp.float32)], axis=1)
        rows = _sc_gather(table, sn, D_TAB)
        worbit = _msg_kernel(rows, rad, sph)
        orbit = _scatter_kernel(starts, sc_row, worbit)
        if r < MP_LOOP:
            dens, coeff = _dens_mlp_kernel(orbit, dens, mp_params[r])

    return _final_kernel(orbit, dens, out_params)


# speedup vs baseline: 19.8174x; 1.0786x over previous
import functools

import jax
import jax.numpy as jnp
import numpy as np
from jax import lax
from jax.experimental import pallas as pl
from jax.experimental.pallas import tpu as pltpu
from jax.experimental.pallas import tpu_sc as plsc

N_ATOMS = 10000
N_EDGES = 320000
NWAVE = 8
NSPH = 9
RL = 3
NORBIT = 24
CUTOFF = 5.0
MP_LOOP = 2

A_BLK = 128
N_PAD = 10240
NB = N_PAD // A_BLK
K_WIN = 512
E_BLK = 2000
D_TAB = 128
D_CART = 128
M_BLK = 1024

_R = np.zeros((NSPH, 72), np.float32)
_T = np.zeros((NWAVE, 72), np.float32)
_P = np.zeros((72, NORBIT), np.float32)
_L_OF_S = [0, 1, 1, 1, 2, 2, 2, 2, 2]
for s in range(NSPH):
    for w in range(NWAVE):
        _R[s, s * 8 + w] = 1.0
        _T[w, s * 8 + w] = 1.0
        _P[s * 8 + w, _L_OF_S[s] * 8 + w] = 1.0


def _sc_gather(table, idx, d):
    info = plsc.get_sparse_core_info()
    nc, ns = info.num_cores, info.num_subcores
    nw = nc * ns
    b = idx.shape[0]
    b_per_w = b // nw
    chunk = 200
    n_chunks = b_per_w // chunk
    mesh = plsc.VectorSubcoreMesh(core_axis_name="c", subcore_axis_name="s")

    @functools.partial(
        pl.kernel, mesh=mesh,
        out_type=jax.ShapeDtypeStruct((b, d), jnp.float32),
        scratch_types=[
            pltpu.VMEM((chunk,), jnp.int32),
            pltpu.VMEM((chunk, d), jnp.float32),
            pltpu.SemaphoreType.DMA,
        ],
    )
    def k(table_hbm, idx_hbm, out_hbm, idx_v, rows_v, sem):
        wid = lax.axis_index("s") * nc + lax.axis_index("c")
        base = wid * b_per_w

        def body(t, carry):
            off = base + t * chunk
            pltpu.sync_copy(idx_hbm.at[pl.ds(off, chunk)], idx_v)
            pltpu.async_copy(table_hbm.at[idx_v], rows_v, sem).wait()
            pltpu.sync_copy(rows_v, out_hbm.at[pl.ds(off, chunk)])
            return carry

        lax.fori_loop(0, n_chunks, body, 0)

    return k(table, idx)


def _geom_kernel(cn, cc, sh, alpha, rs):
    def body(cn_ref, cc_ref, sh_ref, al_ref, rs_ref, rad_ref, sph_ref):
        c = cn_ref[:, 0:3] - cc_ref[:, 0:3] + sh_ref[...]
        x = c[:, 0:1] / CUTOFF
        y = c[:, 1:2] / CUTOFF
        z = c[:, 2:3] / CUTOFF
        r2 = x * x + y * y + z * z
        d = jnp.sqrt(r2) * CUTOFF
        fc = 0.5 * (jnp.cos(np.pi * jnp.clip(d, 0.0, CUTOFF) / CUTOFF) + 1.0)
        g = jnp.exp(-jnp.abs(al_ref[...]) * (d - rs_ref[...]) ** 2)
        rad_ref[...] = g * fc
        sph_ref[...] = jnp.concatenate(
            [jnp.ones_like(x), y, z, x, x * y, y * z, 3.0 * z * z - r2,
             x * z, x * x - y * y], axis=1)

    e = cn.shape[0]
    grid = e // E_BLK
    return pl.pallas_call(
        body,
        grid=(grid,),
        in_specs=[
            pl.BlockSpec((E_BLK, D_CART), lambda i: (i, 0)),
            pl.BlockSpec((E_BLK, D_CART), lambda i: (i, 0)),
            pl.BlockSpec((E_BLK, 3), lambda i: (i, 0)),
            pl.BlockSpec((1, NWAVE), lambda i: (0, 0)),
            pl.BlockSpec((1, NWAVE), lambda i: (0, 0)),
        ],
        out_specs=[
            pl.BlockSpec((E_BLK, NWAVE), lambda i: (i, 0)),
            pl.BlockSpec((E_BLK, NSPH), lambda i: (i, 0)),
        ],
        out_shape=[
            jax.ShapeDtypeStruct((e, NWAVE), jnp.float32),
            jax.ShapeDtypeStruct((e, NSPH), jnp.float32),
        ],
    )(cn, cc, sh, alpha, rs)


def _msg_kernel(rows, rad, sph):
    sel_m = np.zeros((D_TAB, 72), np.float32)
    sel_m[0:72, 0:72] = np.eye(72, dtype=np.float32)
    sel_ct = np.zeros((D_TAB, 72), np.float32)
    for s in range(NSPH):
        for w in range(NWAVE):
            sel_ct[72 + w, s * 8 + w] = 1.0

    def body(rows_ref, rad_ref, sph_ref, r_ref, t_ref, sm_ref, sct_ref,
             out_ref):
        rows_v = rows_ref[...]
        mp72 = jnp.dot(rows_v, sm_ref[...], preferred_element_type=jnp.float32)
        c72 = jnp.dot(rows_v, sct_ref[...], preferred_element_type=jnp.float32)
        s72 = jnp.dot(sph_ref[...], r_ref[...],
                      preferred_element_type=jnp.float32)
        r72 = jnp.dot(rad_ref[...], t_ref[...],
                      preferred_element_type=jnp.float32)
        out_ref[...] = (s72 + mp72) * (r72 * c72)

    e = rows.shape[0]
    grid = e // E_BLK
    return pl.pallas_call(
        body,
        grid=(grid,),
        in_specs=[
            pl.BlockSpec((E_BLK, D_TAB), lambda i: (i, 0)),
            pl.BlockSpec((E_BLK, NWAVE), lambda i: (i, 0)),
            pl.BlockSpec((E_BLK, NSPH), lambda i: (i, 0)),
            pl.BlockSpec((NSPH, 72), lambda i: (0, 0)),
            pl.BlockSpec((NWAVE, 72), lambda i: (0, 0)),
            pl.BlockSpec((D_TAB, 72), lambda i: (0, 0)),
            pl.BlockSpec((D_TAB, 72), lambda i: (0, 0)),
        ],
        out_specs=pl.BlockSpec((E_BLK, 72), lambda i: (i, 0)),
        out_shape=jax.ShapeDtypeStruct((e, 72), jnp.float32),
    )(rows, rad, sph, jnp.asarray(_R), jnp.asarray(_T),
      jnp.asarray(sel_m), jnp.asarray(sel_ct))


def _scatter_kernel(starts, sc_row, worbit):
    def body(starts_ref, sc_hbm, wo_hbm, out_ref, sc_v, wo_v, acc_ref, s1, s2):
        b = pl.program_id(0)
        e0 = starts_ref[b]
        e1 = starts_ref[b + 1]
        t0 = e0 // K_WIN
        t1 = lax.div(e1 + K_WIN - 1, K_WIN)
        rows = lax.broadcasted_iota(jnp.int32, (A_BLK, K_WIN), 0) + b * A_BLK
        acc_ref[...] = jnp.zeros((A_BLK, 72), jnp.float32)

        def win(t, carry):
            cp1 = pltpu.make_async_copy(
                sc_hbm.at[pl.ds(0, 1), pl.ds(t * K_WIN, K_WIN)], sc_v, s1)
            cp2 = pltpu.make_async_copy(
                wo_hbm.at[pl.ds(t * K_WIN, K_WIN), :], wo_v, s2)
            cp1.start()
            cp2.start()
            cp1.wait()
            cp2.wait()
            onehot = jnp.where(rows == sc_v[...], 1.0, 0.0)
            acc_ref[...] += jnp.dot(onehot, wo_v[...],
                                    preferred_element_type=jnp.float32)
            return carry

        lax.fori_loop(t0, t1, win, 0)
        out_ref[...] = acc_ref[...]

    grid_spec = pltpu.PrefetchScalarGridSpec(
        num_scalar_prefetch=1,
        grid=(NB,),
        in_specs=[
            pl.BlockSpec(memory_space=pl.ANY),
            pl.BlockSpec(memory_space=pl.ANY),
        ],
        out_specs=pl.BlockSpec((A_BLK, 72), lambda b, s: (b, 0)),
        scratch_shapes=[
            pltpu.VMEM((1, K_WIN), jnp.int32),
            pltpu.VMEM((K_WIN, 72), jnp.float32),
            pltpu.VMEM((A_BLK, 72), jnp.float32),
            pltpu.SemaphoreType.DMA,
            pltpu.SemaphoreType.DMA,
        ],
    )
    return pl.pallas_call(
        body,
        grid_spec=grid_spec,
        out_shape=jax.ShapeDtypeStruct((N_PAD, 72), jnp.float32),
    )(starts, sc_row, worbit)


def _mlp3(x_ref, w1_ref, b1_ref, w2_ref, b2_ref, w3_ref, b3_ref):
    h = jnp.dot(x_ref[...], w1_ref[...], preferred_element_type=jnp.float32)
    h = jax.nn.silu(h + b1_ref[...])
    h = jnp.dot(h, w2_ref[...], preferred_element_type=jnp.float32)
    h = jax.nn.silu(h + b2_ref[...])
    h = jnp.dot(h, w3_ref[...], preferred_element_type=jnp.float32)
    return h + b3_ref[...]


def _mlp_specs(din, dout):
    return [
        pl.BlockSpec((din, 64), lambda i: (0, 0)),
        pl.BlockSpec((1, 64), lambda i: (0, 0)),
        pl.BlockSpec((64, 64), lambda i: (0, 0)),
        pl.BlockSpec((1, 64), lambda i: (0, 0)),
        pl.BlockSpec((64, dout), lambda i: (0, 0)),
        pl.BlockSpec((1, dout), lambda i: (0, 0)),
    ]


def _emb_kernel(species_pad, params):
    def body(x_ref, w1, b1, w2, b2, w3, b3, out_ref):
        out_ref[...] = _mlp3(x_ref, w1, b1, w2, b2, w3, b3)

    (w1, b1), (w2, b2), (w3, b3) = params
    return pl.pallas_call(
        body,
        grid=(N_PAD // M_BLK,),
        in_specs=[pl.BlockSpec((M_BLK, 1), lambda i: (i, 0))]
        + _mlp_specs(1, NWAVE),
        out_specs=pl.BlockSpec((M_BLK, NWAVE), lambda i: (i, 0)),
        out_shape=jax.ShapeDtypeStruct((N_PAD, NWAVE), jnp.float32),
    )(species_pad, w1, b1[None, :], w2, b2[None, :], w3, b3[None, :])


def _dens_mlp_kernel(orbit, dens, params):
    def body(o_ref, d_ref, p_ref, w1, b1, w2, b2, w3, b3, dn_ref, c_ref):
        o = o_ref[...]
        dn = d_ref[...] + jnp.dot(o * o, p_ref[...],
                                  preferred_element_type=jnp.float32)
        dn_ref[...] = dn
        c_ref[...] = _mlp3(dn_ref, w1, b1, w2, b2, w3, b3)

    (w1, b1), (w2, b2), (w3, b3) = params
    return pl.pallas_call(
        body,
        grid=(N_PAD // M_BLK,),
        in_specs=[
            pl.BlockSpec((M_BLK, 72), lambda i: (i, 0)),
            pl.BlockSpec((M_BLK, NORBIT), lambda i: (i, 0)),
            pl.BlockSpec((72, NORBIT), lambda i: (0, 0)),
        ] + _mlp_specs(NORBIT, NWAVE),
        out_specs=[
            pl.BlockSpec((M_BLK, NORBIT), lambda i: (i, 0)),
            pl.BlockSpec((M_BLK, NWAVE), lambda i: (i, 0)),
        ],
        out_shape=[
            jax.ShapeDtypeStruct((N_PAD, NORBIT), jnp.float32),
            jax.ShapeDtypeStruct((N_PAD, NWAVE), jnp.float32),
        ],
    )(orbit, dens, jnp.asarray(_P), w1, b1[None, :], w2, b2[None, :],
      w3, b3[None, :])


def _final_kernel(orbit, dens, params):
    def body(o_ref, d_ref, p_ref, w1, b1, w2, b2, w3, b3, out_ref, dn_ref):
        i = pl.program_id(0)
        o = o_ref[...]
        dn_ref[...] = d_ref[...] + jnp.dot(
            o * o, p_ref[...], preferred_element_type=jnp.float32)
        y = _mlp3(dn_ref, w1, b1, w2, b2, w3, b3)
        row = lax.broadcasted_iota(jnp.int32, (M_BLK, 1), 0) + i * M_BLK
        y = jnp.where(row < N_ATOMS, y, 0.0)
        s = jnp.sum(y, axis=0, keepdims=True)

        @pl.when(i == 0)
        def _():
            out_ref[...] = jnp.zeros((1, 1), jnp.float32)

        out_ref[...] += s

    (w1, b1), (w2, b2), (w3, b3) = params
    out, _ = pl.pallas_call(
        body,
        grid=(N_PAD // M_BLK,),
        in_specs=[
            pl.BlockSpec((M_BLK, 72), lambda i: (i, 0)),
            pl.BlockSpec((M_BLK, NORBIT), lambda i: (i, 0)),
            pl.BlockSpec((72, NORBIT), lambda i: (0, 0)),
        ] + _mlp_specs(NORBIT, 1),
        out_specs=[
            pl.BlockSpec((1, 1), lambda i: (0, 0)),
            pl.BlockSpec((M_BLK, NORBIT), lambda i: (i, 0)),
        ],
        out_shape=[
            jax.ShapeDtypeStruct((1, 1), jnp.float32),
            jax.ShapeDtypeStruct((N_PAD, NORBIT), jnp.float32),
        ],
    )(orbit, dens, jnp.asarray(_P), w1, b1[None, :], w2, b2[None, :],
      w3, b3[None, :])
    return out[0, 0]


def kernel(cart, shifts, species, radial_params, emb_params, mp_params,
           out_params, atomindex):
    idx_c = atomindex[0]
    idx_n = atomindex[1]

    perm = jnp.argsort(idx_c)
    sc = idx_c[perm]
    sn = idx_n[perm]
    sh = shifts[:, perm].T
    starts = jnp.searchsorted(
        sc, jnp.arange(NB + 1, dtype=jnp.int32) * A_BLK).astype(jnp.int32)
    sc_row = sc[None, :]

    cart_t = jnp.zeros((N_ATOMS, D_CART), jnp.float32)
    cart_t = cart_t.at[:, 0:3].set(cart.T)

    cn = _sc_gather(cart_t, sn, D_CART)
    cc = _sc_gather(cart_t, sc, D_CART)
    alpha, rs = radial_params
    rad, sph = _geom_kernel(cn, cc, sh, alpha[None, :], rs[None, :])

    species_pad = jnp.zeros((N_PAD, 1), jnp.float32).at[:N_ATOMS].set(species)
    coeff = _emb_kernel(species_pad, emb_params)
    dens = jnp.zeros((N_PAD, NORBIT), jnp.float32)
    orbit = jnp.zeros((N_PAD, 72), jnp.float32)

    for r in range(MP_LOOP + 1):
        table = jnp.concatenate(
            [orbit[:N_ATOMS], coeff[:N_ATOMS],
             jnp.zeros((N_ATOMS, D_TAB - 80), jnp.float32)], axis=1)
        rows = _sc_gather(table, sn, D_TAB)
        worbit = _msg_kernel(rows, rad, sph)
        orbit = _scatter_kernel(starts, sc_row, worbit)
        if r < MP_LOOP:
            dens, coeff = _dens_mlp_kernel(orbit, dens, mp_params[r])

    return _final_kernel(orbit, dens, out_params)
